# baseline (speedup 1.0000x reference)
import jax
import jax.numpy as jnp
import numpy as np
from jax.experimental import pallas as pl
from jax.experimental.pallas import tpu as pltpu

N = 10000
DIN = 8
HID = 64
MOUT = 32
ODIM = 8
DT = 0.01
RADIUS = 0.032
R2 = float(np.float32(RADIUS) * np.float32(RADIUS))
CS = 0.032
NC = 32
TILE = 128
CHUNK = 32
NPAD = 10240
NT = NPAD // TILE
NCH = NPAD // CHUNK
NPAIR = NPAD // 2


def _tanh_fast(x):
    s = jnp.clip(x, -7.90531, 7.90531)
    t = s * s
    num = jnp.float32(-2.76076847742355e-16)
    for c in (2.00018790482477e-13, -8.60467152213735e-11,
              5.12229709037114e-08, 1.48572235717979e-05,
              6.37261928875436e-04, 4.89352455891786e-03):
        num = num * t + jnp.float32(c)
    num = s * num
    den = jnp.float32(1.19825839466702e-06)
    for c in (1.18534705686654e-04, 2.26843463243900e-03,
              4.89352518554385e-03):
        den = den * t + jnp.float32(c)
    bits = jax.lax.bitcast_convert_type(den, jnp.int32)
    r = jax.lax.bitcast_convert_type(jnp.int32(0x7EF311C3) - bits,
                                     jnp.float32)
    for _ in range(3):
        r = r * (jnp.float32(2.0) - den * r)
    return num * r


def _precompute_kernel(xs_ref, w1d_ref, w1b_ref, b1_ref, pxr_ref, pyr_ref,
                       pxc_ref, pyc_ref,
                       a_ref, b_ref,
                       tminx_ref, tmaxx_ref, tminy_ref, tmaxy_ref,
                       cminx_ref, cmaxx_ref, cminy_ref, cmaxy_ref):
    xs = xs_ref[...]
    a_ref[...] = jnp.dot(xs, w1d_ref[...],
                         preferred_element_type=jnp.float32) + b1_ref[...]
    b_ref[...] = jnp.dot(xs, w1b_ref[...],
                         preferred_element_type=jnp.float32)
    tminx_ref[...] = jnp.min(pxr_ref[...], axis=1, keepdims=True)
    tmaxx_ref[...] = jnp.max(pxr_ref[...], axis=1, keepdims=True)
    tminy_ref[...] = jnp.min(pyr_ref[...], axis=1, keepdims=True)
    tmaxy_ref[...] = jnp.max(pyr_ref[...], axis=1, keepdims=True)
    cminx_ref[...] = jnp.min(pxc_ref[...], axis=1, keepdims=True)
    cmaxx_ref[...] = jnp.max(pxc_ref[...], axis=1, keepdims=True)
    cminy_ref[...] = jnp.min(pyc_ref[...], axis=1, keepdims=True)
    cmaxy_ref[...] = jnp.max(pyc_ref[...], axis=1, keepdims=True)


def _agg_kernel(a_ref, bp_ref, pxc_ref, pyc_ref,
                tminx_ref, tmaxx_ref, tminy_ref, tmaxy_ref,
                cminx_ref, cmaxx_ref, cminy_ref, cmaxy_ref,
                pxcol_ref, pycol_ref, xc_ref,
                w2_ref, b2_ref, wout_ref, bout_ref,
                out_ref,
                hsum_ref, deg_ref):
    p = pl.program_id(0)
    a = a_ref[...]
    a2 = jnp.concatenate([a, a], axis=1)
    pdx = pxcol_ref[...]
    pdy = pycol_ref[...]
    didx = p * TILE + jax.lax.broadcasted_iota(jnp.int32, (TILE, 1), 0)
    lane64 = jax.lax.broadcasted_iota(jnp.int32, (TILE, TILE), 1) < HID
    hsum_ref[...] = jnp.zeros((TILE, TILE), jnp.float32)
    deg_ref[...] = jnp.zeros((TILE, CHUNK), jnp.float32)
    dlx = jnp.min(pdx)
    dhx = jnp.max(pdx)
    dly = jnp.min(pdy)
    dhy = jnp.max(pdy)

    def tile_body(s, _):
        tgx = jnp.maximum(jnp.maximum(tminx_ref[s, 0] - dhx,
                                      dlx - tmaxx_ref[s, 0]), 0.0)
        tgy = jnp.maximum(jnp.maximum(tminy_ref[s, 0] - dhy,
                                      dly - tmaxy_ref[s, 0]), 0.0)

        @pl.when(tgx * tgx + tgy * tgy <= R2)
        def _tile():
            def chunk_body(c4, _):
                c = s * 4 + c4
                cgx = jnp.maximum(jnp.maximum(cminx_ref[c, 0] - dhx,
                                              dlx - cmaxx_ref[c, 0]), 0.0)
                cgy = jnp.maximum(jnp.maximum(cminy_ref[c, 0] - dhy,
                                              dly - cmaxy_ref[c, 0]), 0.0)

                @pl.when(cgx * cgx + cgy * cgy <= R2)
                def _chunk():
                    sx = pxc_ref[pl.ds(c, 1), :]
                    sy = pyc_ref[pl.ds(c, 1), :]
                    dx = sx - pdx
                    dy = sy - pdy
                    d2 = dx * dx + dy * dy
                    sidx = c * CHUNK + jax.lax.broadcasted_iota(
                        jnp.int32, (1, CHUNK), 1)
                    m = (d2 <= R2) & (sidx != didx) & (sidx < N)
                    mf = m.astype(jnp.float32)
                    deg_ref[...] += mf
                    for k in range(16):
                        mk = mf[:, 2 * k:2 * k + 2]

                        @pl.when(jnp.sum(mk) > 0.0)
                        def _pair(k=k, mf=mf):
                            t = c * 16 + k
                            b2row = bp_ref[pl.ds(t, 1), :]
                            m0 = mf[:, 2 * k:2 * k + 1]
                            m1 = mf[:, 2 * k + 1:2 * k + 2]
                            m2f = jnp.where(lane64, m0, m1)
                            h2 = _tanh_fast(a2 + b2row)
                            hsum_ref[...] += m2f * h2
                return 0

            jax.lax.fori_loop(0, 4, chunk_body, 0)
        return 0

    jax.lax.fori_loop(0, NT, tile_body, 0)

    hs2 = hsum_ref[...]
    hs = hs2[:, :HID] + hs2[:, HID:]
    deg = jnp.sum(deg_ref[...], axis=1, keepdims=True)
    msg = jnp.dot(hs, w2_ref[...],
                  preferred_element_type=jnp.float32) + deg * b2_ref[...]
    msg = msg / jnp.maximum(deg, 1.0)
    outp = jnp.clip(jnp.dot(msg, wout_ref[...],
                            preferred_element_type=jnp.float32) + bout_ref[...],
                    -1.0, 1.0)
    xc = xc_ref[...]
    move = outp[:, 0:2] * DT
    polx = xc[:, 2:3]
    poly = xc[:, 3:4]
    sp_x = xc[:, 0:1] + move[:, 0:1] * polx + move[:, 0:1] * (-poly)
    sp_y = xc[:, 1:2] + move[:, 1:2] * poly + move[:, 1:2] * polx
    rest = xc[:, 2:8] + outp[:, 2:8] * DT
    pvec = rest[:, 0:2]
    nrm = jnp.sqrt(jnp.sum(pvec * pvec, axis=1, keepdims=True))
    pvec = pvec / jnp.maximum(nrm, 1e-8)
    out_ref[...] = jnp.concatenate([sp_x, sp_y, pvec, rest[:, 2:6]], axis=1)


def _step(xc, W1d, W1b, b1_2d, W2, b2_2d, W_out, bout_2d):
    pos = xc[:, :2]
    cell = jnp.clip(jnp.floor(pos / CS).astype(jnp.int32), 0, NC - 1)
    key = cell[:, 1] * NC + cell[:, 0]
    order = jnp.argsort(key)
    xs = jnp.take(xc, order, axis=0)
    xs = jnp.concatenate(
        [xs, jnp.broadcast_to(xs[N - 1:N], (NPAD - N, DIN))], axis=0)
    px = xs[:, 0]
    py = xs[:, 1]
    pxr = px.reshape(NT, TILE)
    pyr = py.reshape(NT, TILE)
    pxc = px.reshape(NCH, CHUNK)
    pyc = py.reshape(NCH, CHUNK)

    f32 = jnp.float32
    pre_out = pl.pallas_call(
        _precompute_kernel,
        out_shape=[
            jax.ShapeDtypeStruct((NPAD, HID), f32),
            jax.ShapeDtypeStruct((NPAD, HID), f32),
            jax.ShapeDtypeStruct((NT, 1), f32),
            jax.ShapeDtypeStruct((NT, 1), f32),
            jax.ShapeDtypeStruct((NT, 1), f32),
            jax.ShapeDtypeStruct((NT, 1), f32),
            jax.ShapeDtypeStruct((NCH, 1), f32),
            jax.ShapeDtypeStruct((NCH, 1), f32),
            jax.ShapeDtypeStruct((NCH, 1), f32),
            jax.ShapeDtypeStruct((NCH, 1), f32),
        ],
    )(xs, W1d, W1b, b1_2d, pxr, pyr, pxc, pyc)
    a, b = pre_out[0], pre_out[1]
    tbb = pre_out[2:6]
    cbb = pre_out[6:10]
    bp = b.reshape(NPAIR, 2 * HID)

    full = lambda shape: pl.BlockSpec(shape, lambda p: (0, 0))
    out_s = pl.pallas_call(
        _agg_kernel,
        grid=(NT,),
        in_specs=[
            pl.BlockSpec((TILE, HID), lambda p: (p, 0)),
            full((NPAIR, 2 * HID)),
            full((NCH, CHUNK)),
            full((NCH, CHUNK)),
            full((NT, 1)), full((NT, 1)),
            full((NT, 1)), full((NT, 1)),
            full((NCH, 1)), full((NCH, 1)),
            full((NCH, 1)), full((NCH, 1)),
            pl.BlockSpec((TILE, 1), lambda p: (p, 0)),
            pl.BlockSpec((TILE, 1), lambda p: (p, 0)),
            pl.BlockSpec((TILE, DIN), lambda p: (p, 0)),
            full((HID, MOUT)),
            full((1, MOUT)),
            full((MOUT, ODIM)),
            full((1, ODIM)),
        ],
        out_specs=pl.BlockSpec((TILE, DIN), lambda p: (p, 0)),
        out_shape=jax.ShapeDtypeStruct((NPAD, DIN), f32),
        scratch_shapes=[
            pltpu.VMEM((TILE, TILE), f32),
            pltpu.VMEM((TILE, CHUNK), f32),
        ],
    )(a, bp, pxc, pyc, *tbb, *cbb,
      px.reshape(NPAD, 1), py.reshape(NPAD, 1), xs,
      W2, b2_2d, W_out, bout_2d)

    new_xs = out_s[:N]
    return jnp.zeros_like(xc).at[order].set(new_xs)


def kernel(x, batch, steps, W1, b1, W2, b2, W_out, b_out):
    x = x.astype(jnp.float32)
    W1d = W1[:DIN] - W1[DIN:]
    W1b = W1[DIN:]
    b1_2d = b1.reshape(1, HID)
    b2_2d = b2.reshape(1, MOUT)
    bout_2d = b_out.reshape(1, ODIM)

    def body(_, xc):
        return _step(xc, W1d, W1b, b1_2d, W2, b2_2d, W_out, bout_2d)

    return jax.lax.fori_loop(0, steps, body, x)


# speedup vs baseline: 2.5586x; 2.5586x over previous
import jax
import jax.numpy as jnp
import numpy as np
from jax.experimental import pallas as pl
from jax.experimental.pallas import tpu as pltpu

N = 10000
DIN = 8
HID = 64
MOUT = 32
ODIM = 8
DT = 0.01
RADIUS = 0.032
R2 = float(np.float32(RADIUS) * np.float32(RADIUS))
CS = 0.032
NC = 32
TILE = 128
CHUNK = 32
NPAD = 10240
NT = NPAD // TILE
NCH = NPAD // CHUNK
NPAIR = NPAD // 2


def _tanh_fast(x):
    s = jnp.clip(x, -7.90531, 7.90531)
    t = s * s
    num = jnp.float32(-2.76076847742355e-16)
    for c in (2.00018790482477e-13, -8.60467152213735e-11,
              5.12229709037114e-08, 1.48572235717979e-05,
              6.37261928875436e-04, 4.89352455891786e-03):
        num = num * t + jnp.float32(c)
    num = s * num
    den = jnp.float32(1.19825839466702e-06)
    for c in (1.18534705686654e-04, 2.26843463243900e-03,
              4.89352518554385e-03):
        den = den * t + jnp.float32(c)
    bits = jax.lax.bitcast_convert_type(den, jnp.int32)
    r = jax.lax.bitcast_convert_type(jnp.int32(0x7EF311C3) - bits,
                                     jnp.float32)
    for _ in range(2):
        r = r * (jnp.float32(2.0) - den * r)
    return num * r


def _precompute_kernel(xs_ref, w1d_ref, w1b_ref, b1_ref, pxr_ref, pyr_ref,
                       pxc_ref, pyc_ref,
                       a_ref, b_ref,
                       tminx_ref, tmaxx_ref, tminy_ref, tmaxy_ref,
                       cminx_ref, cmaxx_ref, cminy_ref, cmaxy_ref):
    xs = xs_ref[...]
    a_ref[...] = jnp.dot(xs, w1d_ref[...],
                         preferred_element_type=jnp.float32) + b1_ref[...]
    b_ref[...] = jnp.dot(xs, w1b_ref[...],
                         preferred_element_type=jnp.float32)
    tminx_ref[...] = jnp.min(pxr_ref[...], axis=1, keepdims=True)
    tmaxx_ref[...] = jnp.max(pxr_ref[...], axis=1, keepdims=True)
    tminy_ref[...] = jnp.min(pyr_ref[...], axis=1, keepdims=True)
    tmaxy_ref[...] = jnp.max(pyr_ref[...], axis=1, keepdims=True)
    cminx_ref[...] = jnp.min(pxc_ref[...], axis=1, keepdims=True)
    cmaxx_ref[...] = jnp.max(pxc_ref[...], axis=1, keepdims=True)
    cminy_ref[...] = jnp.min(pyc_ref[...], axis=1, keepdims=True)
    cmaxy_ref[...] = jnp.max(pyc_ref[...], axis=1, keepdims=True)


def _agg_kernel(a_ref, bp_ref, px2_ref, py2_ref, sidx2_ref,
                tminx_ref, tmaxx_ref, tminy_ref, tmaxy_ref,
                cminx_ref, cmaxx_ref, cminy_ref, cmaxy_ref,
                pxcol_ref, pycol_ref, xc_ref,
                w2_ref, b2_ref, wout_ref, bout_ref,
                out_ref,
                hsum_ref, deg_ref):
    p = pl.program_id(0)
    a = a_ref[...]
    a2 = jnp.concatenate([a, a], axis=1)
    pdx = pxcol_ref[...]
    pdy = pycol_ref[...]
    pdxB = jnp.broadcast_to(pdx, (TILE, TILE))
    pdyB = jnp.broadcast_to(pdy, (TILE, TILE))
    didxB = p * TILE + jax.lax.broadcasted_iota(jnp.int32, (TILE, TILE), 0)
    hsum_ref[...] = jnp.zeros((TILE, TILE), jnp.float32)
    deg_ref[...] = jnp.zeros((TILE, TILE), jnp.float32)
    dlx = jnp.min(pdx)
    dhx = jnp.max(pdx)
    dly = jnp.min(pdy)
    dhy = jnp.max(pdy)

    def tile_body(s, _):
        tgx = jnp.maximum(jnp.maximum(tminx_ref[s, 0] - dhx,
                                      dlx - tmaxx_ref[s, 0]), 0.0)
        tgy = jnp.maximum(jnp.maximum(tminy_ref[s, 0] - dhy,
                                      dly - tmaxy_ref[s, 0]), 0.0)

        @pl.when(tgx * tgx + tgy * tgy <= R2)
        def _tile():
            def chunk_body(c4, _):
                c = s * 4 + c4
                cgx = jnp.maximum(jnp.maximum(cminx_ref[c, 0] - dhx,
                                              dlx - cmaxx_ref[c, 0]), 0.0)
                cgy = jnp.maximum(jnp.maximum(cminy_ref[c, 0] - dhy,
                                              dly - cmaxy_ref[c, 0]), 0.0)

                @pl.when(cgx * cgx + cgy * cgy <= R2)
                def _chunk():
                    base = c * 16
                    for k in range(16):
                        t = base + k
                        b2row = bp_ref[pl.ds(t, 1), :]
                        px2r = px2_ref[pl.ds(t, 1), :]
                        py2r = py2_ref[pl.ds(t, 1), :]
                        sidx2r = sidx2_ref[pl.ds(t, 1), :]
                        dx = px2r - pdxB
                        dy = py2r - pdyB
                        d2 = dx * dx + dy * dy
                        m = (d2 <= R2) & (sidx2r != didxB)
                        mf = jnp.where(m, 1.0, 0.0)
                        h2 = _tanh_fast(a2 + b2row)
                        hsum_ref[...] += mf * h2
                        deg_ref[...] += mf
                return 0

            jax.lax.fori_loop(0, 4, chunk_body, 0)
        return 0

    jax.lax.fori_loop(0, NT, tile_body, 0)

    hs2 = hsum_ref[...]
    hs = hs2[:, :HID] + hs2[:, HID:]
    dg2 = deg_ref[...]
    deg = (dg2[:, :HID] + dg2[:, HID:])[:, 0:1]
    msg = jnp.dot(hs, w2_ref[...],
                  preferred_element_type=jnp.float32) + deg * b2_ref[...]
    msg = msg / jnp.maximum(deg, 1.0)
    outp = jnp.clip(jnp.dot(msg, wout_ref[...],
                            preferred_element_type=jnp.float32) + bout_ref[...],
                    -1.0, 1.0)
    xc = xc_ref[...]
    move = outp[:, 0:2] * DT
    polx = xc[:, 2:3]
    poly = xc[:, 3:4]
    sp_x = xc[:, 0:1] + move[:, 0:1] * polx + move[:, 0:1] * (-poly)
    sp_y = xc[:, 1:2] + move[:, 1:2] * poly + move[:, 1:2] * polx
    rest = xc[:, 2:8] + outp[:, 2:8] * DT
    pvec = rest[:, 0:2]
    nrm = jnp.sqrt(jnp.sum(pvec * pvec, axis=1, keepdims=True))
    pvec = pvec / jnp.maximum(nrm, 1e-8)
    out_ref[...] = jnp.concatenate([sp_x, sp_y, pvec, rest[:, 2:6]], axis=1)


def _step(xc, W1d, W1b, b1_2d, W2, b2_2d, W_out, bout_2d):
    pos = xc[:, :2]
    cell = jnp.clip(jnp.floor(pos / CS).astype(jnp.int32), 0, NC - 1)
    key = cell[:, 1] * NC + cell[:, 0]
    order = jnp.argsort(key)
    xs = jnp.take(xc, order, axis=0)
    xs = jnp.concatenate(
        [xs, jnp.broadcast_to(xs[N - 1:N], (NPAD - N, DIN))], axis=0)
    px = xs[:, 0]
    py = xs[:, 1]
    far = jnp.full((NPAD - N,), 1e9, jnp.float32)
    pxs = jnp.concatenate([px[:N], far])
    pys = jnp.concatenate([py[:N], far])
    pxr = pxs.reshape(NT, TILE)
    pyr = pys.reshape(NT, TILE)
    pxc = pxs.reshape(NCH, CHUNK)
    pyc = pys.reshape(NCH, CHUNK)
    px2 = jnp.broadcast_to(pxs[:, None], (NPAD, HID)).reshape(NPAIR, 2 * HID)
    py2 = jnp.broadcast_to(pys[:, None], (NPAD, HID)).reshape(NPAIR, 2 * HID)
    sidx2 = jnp.broadcast_to(
        jnp.arange(NPAD, dtype=jnp.int32)[:, None],
        (NPAD, HID)).reshape(NPAIR, 2 * HID)

    f32 = jnp.float32
    pre_out = pl.pallas_call(
        _precompute_kernel,
        out_shape=[
            jax.ShapeDtypeStruct((NPAD, HID), f32),
            jax.ShapeDtypeStruct((NPAD, HID), f32),
            jax.ShapeDtypeStruct((NT, 1), f32),
            jax.ShapeDtypeStruct((NT, 1), f32),
            jax.ShapeDtypeStruct((NT, 1), f32),
            jax.ShapeDtypeStruct((NT, 1), f32),
            jax.ShapeDtypeStruct((NCH, 1), f32),
            jax.ShapeDtypeStruct((NCH, 1), f32),
            jax.ShapeDtypeStruct((NCH, 1), f32),
            jax.ShapeDtypeStruct((NCH, 1), f32),
        ],
    )(xs, W1d, W1b, b1_2d, pxr, pyr, pxc, pyc)
    a, b = pre_out[0], pre_out[1]
    tbb = pre_out[2:6]
    cbb = pre_out[6:10]
    bp = b.reshape(NPAIR, 2 * HID)

    full = lambda shape: pl.BlockSpec(shape, lambda p: (0, 0))
    out_s = pl.pallas_call(
        _agg_kernel,
        grid=(NT,),
        in_specs=[
            pl.BlockSpec((TILE, HID), lambda p: (p, 0)),
            full((NPAIR, 2 * HID)),
            full((NPAIR, 2 * HID)),
            full((NPAIR, 2 * HID)),
            full((NPAIR, 2 * HID)),
            full((NT, 1)), full((NT, 1)),
            full((NT, 1)), full((NT, 1)),
            full((NCH, 1)), full((NCH, 1)),
            full((NCH, 1)), full((NCH, 1)),
            pl.BlockSpec((TILE, 1), lambda p: (p, 0)),
            pl.BlockSpec((TILE, 1), lambda p: (p, 0)),
            pl.BlockSpec((TILE, DIN), lambda p: (p, 0)),
            full((HID, MOUT)),
            full((1, MOUT)),
            full((MOUT, ODIM)),
            full((1, ODIM)),
        ],
        out_specs=pl.BlockSpec((TILE, DIN), lambda p: (p, 0)),
        out_shape=jax.ShapeDtypeStruct((NPAD, DIN), f32),
        scratch_shapes=[
            pltpu.VMEM((TILE, TILE), f32),
            pltpu.VMEM((TILE, TILE), f32),
        ],
    )(a, bp, px2, py2, sidx2, *tbb, *cbb,
      px.reshape(NPAD, 1), py.reshape(NPAD, 1), xs,
      W2, b2_2d, W_out, bout_2d)

    new_xs = out_s[:N]
    return jnp.zeros_like(xc).at[order].set(new_xs)


def kernel(x, batch, steps, W1, b1, W2, b2, W_out, b_out):
    x = x.astype(jnp.float32)
    W1d = W1[:DIN] - W1[DIN:]
    W1b = W1[DIN:]
    b1_2d = b1.reshape(1, HID)
    b2_2d = b2.reshape(1, MOUT)
    bout_2d = b_out.reshape(1, ODIM)

    def body(_, xc):
        return _step(xc, W1d, W1b, b1_2d, W2, b2_2d, W_out, bout_2d)

    return jax.lax.fori_loop(0, steps, body, x)


# speedup vs baseline: 5.7855x; 2.2612x over previous
import jax
import jax.numpy as jnp
import numpy as np
from jax.experimental import pallas as pl
from jax.experimental.pallas import tpu as pltpu

N = 10000
DIN = 8
HID = 64
MOUT = 32
ODIM = 8
DT = 0.01
RADIUS = 0.032
R2 = float(np.float32(RADIUS) * np.float32(RADIUS))
CS = 0.032
NC = 32
TILE = 128
CHUNK = 32
NPAD = 10240
NT = NPAD // TILE
NCH = NPAD // CHUNK
NPAIR = NPAD // 2


def _tanh_fast(x):
    s = jnp.clip(x, -7.90531, 7.90531)
    t = s * s
    num = jnp.float32(-2.76076847742355e-16)
    for c in (2.00018790482477e-13, -8.60467152213735e-11,
              5.12229709037114e-08, 1.48572235717979e-05,
              6.37261928875436e-04, 4.89352455891786e-03):
        num = num * t + jnp.float32(c)
    num = s * num
    den = jnp.float32(1.19825839466702e-06)
    for c in (1.18534705686654e-04, 2.26843463243900e-03,
              4.89352518554385e-03):
        den = den * t + jnp.float32(c)
    bits = jax.lax.bitcast_convert_type(den, jnp.int32)
    r = jax.lax.bitcast_convert_type(jnp.int32(0x7EF311C3) - bits,
                                     jnp.float32)
    for _ in range(2):
        r = r * (jnp.float32(2.0) - den * r)
    return num * r


def _precompute_kernel(xs_ref, w1d_ref, w1b_ref, b1_ref, pxr_ref, pyr_ref,
                       pxc_ref, pyc_ref,
                       a_ref, b_ref,
                       tminx_ref, tmaxx_ref, tminy_ref, tmaxy_ref,
                       cminx_ref, cmaxx_ref, cminy_ref, cmaxy_ref):
    xs = xs_ref[...]
    a_ref[...] = jnp.dot(xs, w1d_ref[...],
                         preferred_element_type=jnp.float32) + b1_ref[...]
    b_ref[...] = jnp.dot(xs, w1b_ref[...],
                         preferred_element_type=jnp.float32)
    tminx_ref[...] = jnp.min(pxr_ref[...], axis=1, keepdims=True)
    tmaxx_ref[...] = jnp.max(pxr_ref[...], axis=1, keepdims=True)
    tminy_ref[...] = jnp.min(pyr_ref[...], axis=1, keepdims=True)
    tmaxy_ref[...] = jnp.max(pyr_ref[...], axis=1, keepdims=True)
    cminx_ref[...] = jnp.min(pxc_ref[...], axis=1, keepdims=True)
    cmaxx_ref[...] = jnp.max(pxc_ref[...], axis=1, keepdims=True)
    cminy_ref[...] = jnp.min(pyc_ref[...], axis=1, keepdims=True)
    cmaxy_ref[...] = jnp.max(pyc_ref[...], axis=1, keepdims=True)


def _agg_kernel(a_ref, bp_ref, px2_ref, py2_ref, sidx2_ref,
                tminx_ref, tmaxx_ref, tminy_ref, tmaxy_ref,
                cminx_ref, cmaxx_ref, cminy_ref, cmaxy_ref,
                pxcol_ref, pycol_ref, xc_ref,
                w2_ref, b2_ref, wout_ref, bout_ref,
                out_ref,
                hsum_ref, deg_ref):
    p = pl.program_id(0)
    a = a_ref[...]
    a2 = jnp.concatenate([a, a], axis=1)
    pdx = pxcol_ref[...]
    pdy = pycol_ref[...]
    pdxB = jnp.broadcast_to(pdx, (TILE, TILE))
    pdyB = jnp.broadcast_to(pdy, (TILE, TILE))
    didxB = p * TILE + jax.lax.broadcasted_iota(jnp.int32, (TILE, TILE), 0)
    hsum_ref[...] = jnp.zeros((TILE, TILE), jnp.float32)
    deg_ref[...] = jnp.zeros((TILE, TILE), jnp.float32)
    dlx = jnp.min(pdx)
    dhx = jnp.max(pdx)
    dly = jnp.min(pdy)
    dhy = jnp.max(pdy)

    def tile_body(s, _):
        tgx = jnp.maximum(jnp.maximum(tminx_ref[s, 0] - dhx,
                                      dlx - tmaxx_ref[s, 0]), 0.0)
        tgy = jnp.maximum(jnp.maximum(tminy_ref[s, 0] - dhy,
                                      dly - tmaxy_ref[s, 0]), 0.0)

        @pl.when(tgx * tgx + tgy * tgy <= R2)
        def _tile():
            def chunk_body(c4, _):
                c = s * 4 + c4
                cgx = jnp.maximum(jnp.maximum(cminx_ref[c, 0] - dhx,
                                              dlx - cmaxx_ref[c, 0]), 0.0)
                cgy = jnp.maximum(jnp.maximum(cminy_ref[c, 0] - dhy,
                                              dly - cmaxy_ref[c, 0]), 0.0)

                @pl.when(cgx * cgx + cgy * cgy <= R2)
                def _chunk():
                    base = c * 16
                    for k in range(16):
                        t = base + k
                        b2row = bp_ref[pl.ds(t, 1), :]
                        px2r = px2_ref[pl.ds(t, 1), :]
                        py2r = py2_ref[pl.ds(t, 1), :]
                        sidx2r = sidx2_ref[pl.ds(t, 1), :]
                        dx = px2r - pdxB
                        dy = py2r - pdyB
                        d2 = dx * dx + dy * dy
                        m = (d2 <= R2) & (sidx2r != didxB)
                        mf = jnp.where(m, 1.0, 0.0)
                        h2 = jnp.tanh(a2 + b2row)
                        hsum_ref[...] += mf * h2
                        deg_ref[...] += mf
                return 0

            jax.lax.fori_loop(0, 4, chunk_body, 0)
        return 0

    jax.lax.fori_loop(0, NT, tile_body, 0)

    hs2 = hsum_ref[...]
    hs = hs2[:, :HID] + hs2[:, HID:]
    dg2 = deg_ref[...]
    deg = (dg2[:, :HID] + dg2[:, HID:])[:, 0:1]
    msg = jnp.dot(hs, w2_ref[...],
                  preferred_element_type=jnp.float32) + deg * b2_ref[...]
    msg = msg / jnp.maximum(deg, 1.0)
    outp = jnp.clip(jnp.dot(msg, wout_ref[...],
                            preferred_element_type=jnp.float32) + bout_ref[...],
                    -1.0, 1.0)
    xc = xc_ref[...]
    move = outp[:, 0:2] * DT
    polx = xc[:, 2:3]
    poly = xc[:, 3:4]
    sp_x = xc[:, 0:1] + move[:, 0:1] * polx + move[:, 0:1] * (-poly)
    sp_y = xc[:, 1:2] + move[:, 1:2] * poly + move[:, 1:2] * polx
    rest = xc[:, 2:8] + outp[:, 2:8] * DT
    pvec = rest[:, 0:2]
    nrm = jnp.sqrt(jnp.sum(pvec * pvec, axis=1, keepdims=True))
    pvec = pvec / jnp.maximum(nrm, 1e-8)
    out_ref[...] = jnp.concatenate([sp_x, sp_y, pvec, rest[:, 2:6]], axis=1)


def _step(xc, W1d, W1b, b1_2d, W2, b2_2d, W_out, bout_2d):
    pos = xc[:, :2]
    cell = jnp.clip(jnp.floor(pos / CS).astype(jnp.int32), 0, NC - 1)
    key = cell[:, 1] * NC + cell[:, 0]
    order = jnp.argsort(key)
    xs = jnp.take(xc, order, axis=0)
    xs = jnp.concatenate(
        [xs, jnp.broadcast_to(xs[N - 1:N], (NPAD - N, DIN))], axis=0)
    px = xs[:, 0]
    py = xs[:, 1]
    far = jnp.full((NPAD - N,), 1e9, jnp.float32)
    pxs = jnp.concatenate([px[:N], far])
    pys = jnp.concatenate([py[:N], far])
    pxr = pxs.reshape(NT, TILE)
    pyr = pys.reshape(NT, TILE)
    pxc = pxs.reshape(NCH, CHUNK)
    pyc = pys.reshape(NCH, CHUNK)
    px2 = jnp.broadcast_to(pxs[:, None], (NPAD, HID)).reshape(NPAIR, 2 * HID)
    py2 = jnp.broadcast_to(pys[:, None], (NPAD, HID)).reshape(NPAIR, 2 * HID)
    sidx2 = jnp.broadcast_to(
        jnp.arange(NPAD, dtype=jnp.int32)[:, None],
        (NPAD, HID)).reshape(NPAIR, 2 * HID)

    f32 = jnp.float32
    pre_out = pl.pallas_call(
        _precompute_kernel,
        out_shape=[
            jax.ShapeDtypeStruct((NPAD, HID), f32),
            jax.ShapeDtypeStruct((NPAD, HID), f32),
            jax.ShapeDtypeStruct((NT, 1), f32),
            jax.ShapeDtypeStruct((NT, 1), f32),
            jax.ShapeDtypeStruct((NT, 1), f32),
            jax.ShapeDtypeStruct((NT, 1), f32),
            jax.ShapeDtypeStruct((NCH, 1), f32),
            jax.ShapeDtypeStruct((NCH, 1), f32),
            jax.ShapeDtypeStruct((NCH, 1), f32),
            jax.ShapeDtypeStruct((NCH, 1), f32),
        ],
    )(xs, W1d, W1b, b1_2d, pxr, pyr, pxc, pyc)
    a, b = pre_out[0], pre_out[1]
    tbb = pre_out[2:6]
    cbb = pre_out[6:10]
    bp = b.reshape(NPAIR, 2 * HID)

    full = lambda shape: pl.BlockSpec(shape, lambda p: (0, 0))
    out_s = pl.pallas_call(
        _agg_kernel,
        grid=(NT,),
        in_specs=[
            pl.BlockSpec((TILE, HID), lambda p: (p, 0)),
            full((NPAIR, 2 * HID)),
            full((NPAIR, 2 * HID)),
            full((NPAIR, 2 * HID)),
            full((NPAIR, 2 * HID)),
            full((NT, 1)), full((NT, 1)),
            full((NT, 1)), full((NT, 1)),
            full((NCH, 1)), full((NCH, 1)),
            full((NCH, 1)), full((NCH, 1)),
            pl.BlockSpec((TILE, 1), lambda p: (p, 0)),
            pl.BlockSpec((TILE, 1), lambda p: (p, 0)),
            pl.BlockSpec((TILE, DIN), lambda p: (p, 0)),
            full((HID, MOUT)),
            full((1, MOUT)),
            full((MOUT, ODIM)),
            full((1, ODIM)),
        ],
        out_specs=pl.BlockSpec((TILE, DIN), lambda p: (p, 0)),
        out_shape=jax.ShapeDtypeStruct((NPAD, DIN), f32),
        scratch_shapes=[
            pltpu.VMEM((TILE, TILE), f32),
            pltpu.VMEM((TILE, TILE), f32),
        ],
    )(a, bp, px2, py2, sidx2, *tbb, *cbb,
      px.reshape(NPAD, 1), py.reshape(NPAD, 1), xs,
      W2, b2_2d, W_out, bout_2d)

    new_xs = out_s[:N]
    return jnp.zeros_like(xc).at[order].set(new_xs)


def kernel(x, batch, steps, W1, b1, W2, b2, W_out, b_out):
    x = x.astype(jnp.float32)
    W1d = W1[:DIN] - W1[DIN:]
    W1b = W1[DIN:]
    b1_2d = b1.reshape(1, HID)
    b2_2d = b2.reshape(1, MOUT)
    bout_2d = b_out.reshape(1, ODIM)

    def body(_, xc):
        return _step(xc, W1d, W1b, b1_2d, W2, b2_2d, W_out, bout_2d)

    return jax.lax.fori_loop(0, steps, body, x)


# speedup vs baseline: 5.9742x; 1.0326x over previous
import jax
import jax.numpy as jnp
import numpy as np
from jax.experimental import pallas as pl
from jax.experimental.pallas import tpu as pltpu

N = 10000
DIN = 8
HID = 64
MOUT = 32
ODIM = 8
DT = 0.01
RADIUS = 0.032
R2 = float(np.float32(RADIUS) * np.float32(RADIUS))
CS = 0.032
NC = 32
TILE = 128
CHUNK = 32
NPAD = 10240
NT = NPAD // TILE
NCH = NPAD // CHUNK
NPAIR = NPAD // 2


def _tanh_fast(x):
    s = jnp.clip(x, -7.90531, 7.90531)
    t = s * s
    num = jnp.float32(-2.76076847742355e-16)
    for c in (2.00018790482477e-13, -8.60467152213735e-11,
              5.12229709037114e-08, 1.48572235717979e-05,
              6.37261928875436e-04, 4.89352455891786e-03):
        num = num * t + jnp.float32(c)
    num = s * num
    den = jnp.float32(1.19825839466702e-06)
    for c in (1.18534705686654e-04, 2.26843463243900e-03,
              4.89352518554385e-03):
        den = den * t + jnp.float32(c)
    bits = jax.lax.bitcast_convert_type(den, jnp.int32)
    r = jax.lax.bitcast_convert_type(jnp.int32(0x7EF311C3) - bits,
                                     jnp.float32)
    for _ in range(2):
        r = r * (jnp.float32(2.0) - den * r)
    return num * r


def _precompute_kernel(xs_ref, w1d_ref, w1b_ref, b1_ref, pxr_ref, pyr_ref,
                       pxc_ref, pyc_ref,
                       a_ref, b_ref,
                       tminx_ref, tmaxx_ref, tminy_ref, tmaxy_ref,
                       cminx_ref, cmaxx_ref, cminy_ref, cmaxy_ref):
    xs = xs_ref[...]
    a_ref[...] = jnp.dot(xs, w1d_ref[...],
                         preferred_element_type=jnp.float32) + b1_ref[...]
    b_ref[...] = jnp.dot(xs, w1b_ref[...],
                         preferred_element_type=jnp.float32)
    tminx_ref[...] = jnp.min(pxr_ref[...], axis=1, keepdims=True)
    tmaxx_ref[...] = jnp.max(pxr_ref[...], axis=1, keepdims=True)
    tminy_ref[...] = jnp.min(pyr_ref[...], axis=1, keepdims=True)
    tmaxy_ref[...] = jnp.max(pyr_ref[...], axis=1, keepdims=True)
    cminx_ref[...] = jnp.min(pxc_ref[...], axis=1, keepdims=True)
    cmaxx_ref[...] = jnp.max(pxc_ref[...], axis=1, keepdims=True)
    cminy_ref[...] = jnp.min(pyc_ref[...], axis=1, keepdims=True)
    cmaxy_ref[...] = jnp.max(pyc_ref[...], axis=1, keepdims=True)


def _agg_kernel(a_ref, bp_ref, px2_ref, py2_ref, sidx2_ref,
                tminx_ref, tmaxx_ref, tminy_ref, tmaxy_ref,
                cminx_ref, cmaxx_ref, cminy_ref, cmaxy_ref,
                pxcol_ref, pycol_ref, xc_ref,
                w2_ref, b2_ref, wout_ref, bout_ref,
                out_ref,
                hsum_ref, deg_ref):
    p = pl.program_id(0)
    a = a_ref[...]
    a2 = jnp.concatenate([a, a], axis=1)
    pdx = pxcol_ref[...]
    pdy = pycol_ref[...]
    pdxB = jnp.broadcast_to(pdx, (TILE, TILE))
    pdyB = jnp.broadcast_to(pdy, (TILE, TILE))
    didxB = p * TILE + jax.lax.broadcasted_iota(jnp.int32, (TILE, TILE), 0)
    hsum_ref[...] = jnp.zeros((TILE, TILE), jnp.float32)
    deg_ref[...] = jnp.zeros((TILE, TILE), jnp.float32)
    dlx = jnp.min(pdx)
    dhx = jnp.max(pdx)
    dly = jnp.min(pdy)
    dhy = jnp.max(pdy)

    def tile_body(s, _):
        tgx = jnp.maximum(jnp.maximum(tminx_ref[s, 0] - dhx,
                                      dlx - tmaxx_ref[s, 0]), 0.0)
        tgy = jnp.maximum(jnp.maximum(tminy_ref[s, 0] - dhy,
                                      dly - tmaxy_ref[s, 0]), 0.0)

        @pl.when(tgx * tgx + tgy * tgy <= R2)
        def _tile():
            def make_chunk_body(check_self):
                def chunk_body(c4, _):
                    c = s * 4 + c4
                    cgx = jnp.maximum(jnp.maximum(cminx_ref[c, 0] - dhx,
                                                  dlx - cmaxx_ref[c, 0]), 0.0)
                    cgy = jnp.maximum(jnp.maximum(cminy_ref[c, 0] - dhy,
                                                  dly - cmaxy_ref[c, 0]), 0.0)

                    @pl.when(cgx * cgx + cgy * cgy <= R2)
                    def _chunk():
                        base = c * 16
                        acc_h = None
                        acc_d = None
                        for k in range(16):
                            t = base + k
                            b2row = bp_ref[pl.ds(t, 1), :]
                            px2r = px2_ref[pl.ds(t, 1), :]
                            py2r = py2_ref[pl.ds(t, 1), :]
                            dx = px2r - pdxB
                            dy = py2r - pdyB
                            d2 = dx * dx + dy * dy
                            m = d2 <= R2
                            if check_self:
                                sidx2r = sidx2_ref[pl.ds(t, 1), :]
                                m = m & (sidx2r != didxB)
                            mf = jnp.where(m, 1.0, 0.0)
                            h2 = jnp.tanh(a2 + b2row)
                            hk = mf * h2
                            acc_h = hk if acc_h is None else acc_h + hk
                            acc_d = mf if acc_d is None else acc_d + mf
                        hsum_ref[...] += acc_h
                        deg_ref[...] += acc_d
                    return 0
                return chunk_body

            @pl.when(s == p)
            def _diag():
                jax.lax.fori_loop(0, 4, make_chunk_body(True), 0)

            @pl.when(s != p)
            def _offdiag():
                jax.lax.fori_loop(0, 4, make_chunk_body(False), 0)
        return 0

    jax.lax.fori_loop(0, NT, tile_body, 0)

    hs2 = hsum_ref[...]
    hs = hs2[:, :HID] + hs2[:, HID:]
    dg2 = deg_ref[...]
    deg = (dg2[:, :HID] + dg2[:, HID:])[:, 0:1]
    msg = jnp.dot(hs, w2_ref[...],
                  preferred_element_type=jnp.float32) + deg * b2_ref[...]
    msg = msg / jnp.maximum(deg, 1.0)
    outp = jnp.clip(jnp.dot(msg, wout_ref[...],
                            preferred_element_type=jnp.float32) + bout_ref[...],
                    -1.0, 1.0)
    xc = xc_ref[...]
    move = outp[:, 0:2] * DT
    polx = xc[:, 2:3]
    poly = xc[:, 3:4]
    sp_x = xc[:, 0:1] + move[:, 0:1] * polx + move[:, 0:1] * (-poly)
    sp_y = xc[:, 1:2] + move[:, 1:2] * poly + move[:, 1:2] * polx
    rest = xc[:, 2:8] + outp[:, 2:8] * DT
    pvec = rest[:, 0:2]
    nrm = jnp.sqrt(jnp.sum(pvec * pvec, axis=1, keepdims=True))
    pvec = pvec / jnp.maximum(nrm, 1e-8)
    out_ref[...] = jnp.concatenate([sp_x, sp_y, pvec, rest[:, 2:6]], axis=1)


def _step(xc, W1d, W1b, b1_2d, W2, b2_2d, W_out, bout_2d):
    pos = xc[:, :2]
    cell = jnp.clip(jnp.floor(pos / CS).astype(jnp.int32), 0, NC - 1)
    key = cell[:, 1] * NC + cell[:, 0]
    order = jnp.argsort(key)
    xs = jnp.take(xc, order, axis=0)
    xs = jnp.concatenate(
        [xs, jnp.broadcast_to(xs[N - 1:N], (NPAD - N, DIN))], axis=0)
    px = xs[:, 0]
    py = xs[:, 1]
    far = jnp.full((NPAD - N,), 1e9, jnp.float32)
    pxs = jnp.concatenate([px[:N], far])
    pys = jnp.concatenate([py[:N], far])
    pxr = pxs.reshape(NT, TILE)
    pyr = pys.reshape(NT, TILE)
    pxc = pxs.reshape(NCH, CHUNK)
    pyc = pys.reshape(NCH, CHUNK)
    px2 = jnp.broadcast_to(pxs[:, None], (NPAD, HID)).reshape(NPAIR, 2 * HID)
    py2 = jnp.broadcast_to(pys[:, None], (NPAD, HID)).reshape(NPAIR, 2 * HID)
    sidx2 = jnp.broadcast_to(
        jnp.arange(NPAD, dtype=jnp.int32)[:, None],
        (NPAD, HID)).reshape(NPAIR, 2 * HID)

    f32 = jnp.float32
    pre_out = pl.pallas_call(
        _precompute_kernel,
        out_shape=[
            jax.ShapeDtypeStruct((NPAD, HID), f32),
            jax.ShapeDtypeStruct((NPAD, HID), f32),
            jax.ShapeDtypeStruct((NT, 1), f32),
            jax.ShapeDtypeStruct((NT, 1), f32),
            jax.ShapeDtypeStruct((NT, 1), f32),
            jax.ShapeDtypeStruct((NT, 1), f32),
            jax.ShapeDtypeStruct((NCH, 1), f32),
            jax.ShapeDtypeStruct((NCH, 1), f32),
            jax.ShapeDtypeStruct((NCH, 1), f32),
            jax.ShapeDtypeStruct((NCH, 1), f32),
        ],
    )(xs, W1d, W1b, b1_2d, pxr, pyr, pxc, pyc)
    a, b = pre_out[0], pre_out[1]
    tbb = pre_out[2:6]
    cbb = pre_out[6:10]
    bp = b.reshape(NPAIR, 2 * HID)

    full = lambda shape: pl.BlockSpec(shape, lambda p: (0, 0))
    out_s = pl.pallas_call(
        _agg_kernel,
        grid=(NT,),
        in_specs=[
            pl.BlockSpec((TILE, HID), lambda p: (p, 0)),
            full((NPAIR, 2 * HID)),
            full((NPAIR, 2 * HID)),
            full((NPAIR, 2 * HID)),
            full((NPAIR, 2 * HID)),
            full((NT, 1)), full((NT, 1)),
            full((NT, 1)), full((NT, 1)),
            full((NCH, 1)), full((NCH, 1)),
            full((NCH, 1)), full((NCH, 1)),
            pl.BlockSpec((TILE, 1), lambda p: (p, 0)),
            pl.BlockSpec((TILE, 1), lambda p: (p, 0)),
            pl.BlockSpec((TILE, DIN), lambda p: (p, 0)),
            full((HID, MOUT)),
            full((1, MOUT)),
            full((MOUT, ODIM)),
            full((1, ODIM)),
        ],
        out_specs=pl.BlockSpec((TILE, DIN), lambda p: (p, 0)),
        out_shape=jax.ShapeDtypeStruct((NPAD, DIN), f32),
        scratch_shapes=[
            pltpu.VMEM((TILE, TILE), f32),
            pltpu.VMEM((TILE, TILE), f32),
        ],
    )(a, bp, px2, py2, sidx2, *tbb, *cbb,
      px.reshape(NPAD, 1), py.reshape(NPAD, 1), xs,
      W2, b2_2d, W_out, bout_2d)

    new_xs = out_s[:N]
    return jnp.zeros_like(xc).at[order].set(new_xs)


def kernel(x, batch, steps, W1, b1, W2, b2, W_out, b_out):
    x = x.astype(jnp.float32)
    W1d = W1[:DIN] - W1[DIN:]
    W1b = W1[DIN:]
    b1_2d = b1.reshape(1, HID)
    b2_2d = b2.reshape(1, MOUT)
    bout_2d = b_out.reshape(1, ODIM)

    def body(_, xc):
        return _step(xc, W1d, W1b, b1_2d, W2, b2_2d, W_out, bout_2d)

    return jax.lax.fori_loop(0, steps, body, x)
